# baseline (speedup 1.0000x reference)
import functools

import jax
import jax.numpy as jnp
from jax import lax
from jax.experimental import pallas as pl
from jax.experimental.pallas import tpu as pltpu
from jax.experimental.pallas import tpu_sc as plsc

B = 16384
D = 128
NC = 2
NS = 16
NW = NC * NS
NCHUNKS = 2
CB = B // NCHUNKS
BPW = CB // NW
CH = 128
NSTREAM = BPW // CH
BLK = 4096


def _gather_sc(idx2d, emb_table):
    mesh = plsc.VectorSubcoreMesh(core_axis_name="c", subcore_axis_name="s")

    @functools.partial(
        pl.kernel,
        mesh=mesh,
        out_type=jax.ShapeDtypeStruct((CB, D), jnp.float32),
        scratch_types=[
            pltpu.VMEM((NSTREAM, CH), jnp.int32),
            pltpu.VMEM((BPW, D), jnp.float32),
            pltpu.SemaphoreType.DMA,
        ],
    )
    def k(idx_hbm, table_hbm, out_hbm, idx_v, rows_v, sem):
        wid = lax.axis_index("s") * NC + lax.axis_index("c")
        pltpu.sync_copy(idx_hbm.at[pl.ds(wid * NSTREAM, NSTREAM)], idx_v)
        copies = [
            pltpu.async_copy(
                table_hbm.at[idx_v.at[j]], rows_v.at[pl.ds(j * CH, CH)], sem
            )
            for j in range(NSTREAM)
        ]
        for c in copies:
            c.wait()
        pltpu.sync_copy(rows_v, out_hbm.at[pl.ds(wid * BPW, BPW)])

    return k(idx2d, emb_table)


def _mlp_body(x_ref, *refs):
    w1_ref, s1_ref, c1_ref, w2_ref, b2_ref, o_ref = refs[-6:]
    x = x_ref[...]
    m = jnp.mean(x, axis=-1, keepdims=True)
    q = jnp.mean(x * x, axis=-1, keepdims=True)
    rstd = lax.rsqrt(q - m * m + 1e-5)
    p = jnp.dot(x, w1_ref[...], preferred_element_type=jnp.float32)
    h = rstd * (p - m * s1_ref[...]) + c1_ref[...]
    h = h * jax.nn.sigmoid(h)
    o_ref[...] = jnp.dot(h, w2_ref[...],
                         preferred_element_type=jnp.float32) + b2_ref[...]


def _mlp_tc(x, out_buf, chunk, W1g, s1, c1, W2, b22):
    base_blk = chunk * (CB // BLK)
    vec = pl.BlockSpec((1, D), lambda i: (0, 0))
    mat = pl.BlockSpec((D, D), lambda i: (0, 0))
    weight_specs = [mat, vec, vec, mat, vec]
    if out_buf is None:
        in_specs = [pl.BlockSpec((BLK, D), lambda i: (i, 0))] + weight_specs
        args = (x, W1g, s1, c1, W2, b22)
        aliases = {}
    else:
        in_specs = [pl.BlockSpec((BLK, D), lambda i: (i, 0)),
                    pl.BlockSpec(memory_space=pl.ANY)] + weight_specs
        args = (x, out_buf, W1g, s1, c1, W2, b22)
        aliases = {1: 0}
    return pl.pallas_call(
        _mlp_body,
        grid=(CB // BLK,),
        in_specs=in_specs,
        out_specs=pl.BlockSpec((BLK, D), lambda i: (i + base_blk, 0)),
        out_shape=jax.ShapeDtypeStruct((B, D), jnp.float32),
        input_output_aliases=aliases,
    )(*args)


def kernel(classes, cond_drop_prob, emb_table, null_classes_emb,
           ln_gamma, ln_beta, W1, b1, W2, b2):
    W1g = ln_gamma[:, None] * W1
    s1 = jnp.sum(W1g, axis=0).reshape(1, D)
    c1 = (ln_beta @ W1 + b1).reshape(1, D)
    b22 = b2.reshape(1, D)
    idx3d = classes.reshape(NCHUNKS, NW * NSTREAM, CH)
    embs = [_gather_sc(idx3d[c], emb_table) for c in range(NCHUNKS)]
    out = None
    for c in range(NCHUNKS):
        out = _mlp_tc(embs[c], out, c, W1g, s1, c1, W2, b22)
    return out
